# baseline (speedup 1.0000x reference)
import jax
import jax.numpy as jnp
from jax.experimental import pallas as pl
from jax.experimental.pallas import tpu as pltpu


def _lenet_body(x_ref, t1_ref, c1b_ref, t2_ref, c2b_ref,
                w1_ref, b1_ref, w2_ref, b2_ref, w3_ref, b3_ref, o_ref):
    B = x_ref.shape[0]
    x = x_ref[...]

    acc = jnp.zeros((B * 28, 256), jnp.float32)
    for kh in range(5):
        xs = x[:, kh:kh + 28, :].reshape(B * 28, 96)
        acc = acc + jnp.dot(xs, t1_ref[kh],
                            preferred_element_type=jnp.float32)
    y = jnp.maximum(acc + c1b_ref[...], 0.0).astype(jnp.bfloat16)
    y = y.reshape(B, 14, 2, 256)
    y = jnp.maximum(y[:, :, 0, :], y[:, :, 1, :])
    x2 = jnp.maximum(y[:, :, :128], y[:, :, 128:])

    acc2 = jnp.zeros((B * 10, 256), jnp.float32)
    for kh in range(5):
        xs2 = x2[:, kh:kh + 10, :].reshape(B * 10, 128)
        acc2 = acc2 + jnp.dot(xs2, t2_ref[kh],
                              preferred_element_type=jnp.float32)
    y2 = jnp.maximum(acc2 + c2b_ref[...], 0.0).astype(jnp.bfloat16)
    y2 = y2.reshape(B, 5, 2, 256)
    y2 = jnp.maximum(y2[:, :, 0, :], y2[:, :, 1, :])
    feat = jnp.maximum(y2[:, :, :128], y2[:, :, 128:])

    h = jnp.zeros((B, 128), jnp.float32)
    for r in range(5):
        h = h + jnp.dot(feat[:, r, :], w1_ref[r],
                        preferred_element_type=jnp.float32)
    h = jnp.maximum(h + b1_ref[...], 0.0).astype(jnp.bfloat16)
    h = jnp.dot(h, w2_ref[...], preferred_element_type=jnp.float32)
    h = jnp.maximum(h + b2_ref[...], 0.0).astype(jnp.bfloat16)
    h = jnp.dot(h, w3_ref[...], preferred_element_type=jnp.float32)
    o_ref[...] = h + b3_ref[...]


def _build_tables(conv1_w, conv1_b, conv2_w, conv2_b,
                  fc1_w, fc1_b, fc2_w, fc2_b, fc3_w, fc3_b):
    f32 = jnp.float32

    wt1 = conv1_w.astype(f32).transpose(1, 2, 3, 0)
    ow = jnp.arange(28)
    t1 = jnp.zeros((5, 32, 3, 2, 14, 6), f32)
    for kw in range(5):
        vals = jnp.broadcast_to(wt1[:, :, kw, :].transpose(1, 0, 2)[None],
                                (28, 5, 3, 6))
        t1 = t1.at[:, ow + kw, :, ow % 2, ow // 2, :].set(vals)
    t1 = jnp.pad(t1.reshape(5, 96, 2, 84), ((0, 0), (0, 0), (0, 0), (0, 44)))
    t1 = t1.reshape(5, 96, 256)

    wt2 = conv2_w.astype(f32).transpose(1, 2, 3, 0)
    ow2 = jnp.arange(10)
    t2 = jnp.zeros((5, 14, 6, 2, 5, 16), f32)
    for kw in range(5):
        vals = jnp.broadcast_to(wt2[:, :, kw, :].transpose(1, 0, 2)[None],
                                (10, 5, 6, 16))
        t2 = t2.at[:, ow2 + kw, :, ow2 % 2, ow2 // 2, :].set(vals)
    t2 = jnp.pad(t2.reshape(5, 84, 2, 80),
                 ((0, 0), (0, 44), (0, 0), (0, 48)))
    t2 = t2.reshape(5, 128, 256)

    half1 = jnp.pad(jnp.tile(conv1_b.astype(f32), 14), (0, 44))
    c1b = jnp.concatenate([half1, half1]).reshape(1, 256)
    half2 = jnp.pad(jnp.tile(conv2_b.astype(f32), 5), (0, 48))
    c2b = jnp.concatenate([half2, half2]).reshape(1, 256)

    w1 = (fc1_w.astype(f32).reshape(16, 5, 5, 120)
          .transpose(1, 2, 0, 3).reshape(5, 80, 120))
    w1 = jnp.pad(w1, ((0, 0), (0, 48), (0, 8)))
    b1 = jnp.pad(fc1_b.astype(f32), (0, 8)).reshape(1, 128)
    w2 = jnp.pad(fc2_w.astype(f32), ((0, 8), (0, 44)))
    b2 = jnp.pad(fc2_b.astype(f32), (0, 44)).reshape(1, 128)
    w3 = jnp.pad(fc3_w.astype(f32), ((0, 44), (0, 118)))
    b3 = jnp.pad(fc3_b.astype(f32), (0, 118)).reshape(1, 128)
    bf16 = jnp.bfloat16
    return (t1.astype(bf16), c1b, t2.astype(bf16), c2b,
            w1.astype(bf16), b1, w2.astype(bf16), b2, w3.astype(bf16), b3)


def kernel(x, conv1_w, conv1_b, conv2_w, conv2_b,
           fc1_w, fc1_b, fc2_w, fc2_b, fc3_w, fc3_b):
    N = x.shape[0]
    B = 256
    while N % B:
        B //= 2
    tables = _build_tables(conv1_w, conv1_b, conv2_w, conv2_b,
                           fc1_w, fc1_b, fc2_w, fc2_b, fc3_w, fc3_b)
    xt = x.astype(jnp.bfloat16).transpose(0, 2, 3, 1).reshape(N, 32, 96)
    out = pl.pallas_call(
        _lenet_body,
        out_shape=jax.ShapeDtypeStruct((N, 128), jnp.float32),
        grid=(N // B,),
        in_specs=[
            pl.BlockSpec((B, 32, 96), lambda i: (i, 0, 0)),
            pl.BlockSpec((5, 96, 256), lambda i: (0, 0, 0)),
            pl.BlockSpec((1, 256), lambda i: (0, 0)),
            pl.BlockSpec((5, 128, 256), lambda i: (0, 0, 0)),
            pl.BlockSpec((1, 256), lambda i: (0, 0)),
            pl.BlockSpec((5, 128, 128), lambda i: (0, 0, 0)),
            pl.BlockSpec((1, 128), lambda i: (0, 0)),
            pl.BlockSpec((128, 128), lambda i: (0, 0)),
            pl.BlockSpec((1, 128), lambda i: (0, 0)),
            pl.BlockSpec((128, 128), lambda i: (0, 0)),
            pl.BlockSpec((1, 128), lambda i: (0, 0)),
        ],
        out_specs=pl.BlockSpec((B, 128), lambda i: (i, 0)),
        compiler_params=pltpu.CompilerParams(
            dimension_semantics=("parallel",),
            vmem_limit_bytes=100 * 1024 * 1024,
        ),
        cost_estimate=pl.CostEstimate(
            flops=16_000_000_000,
            transcendentals=0,
            bytes_accessed=x.size * 4 + N * 128 * 4,
        ),
    )(xt, *tables)
    return out[:, :10]


# speedup vs baseline: 1.2062x; 1.2062x over previous
import jax
import jax.numpy as jnp
from jax.experimental import pallas as pl
from jax.experimental.pallas import tpu as pltpu


def _lenet_body(x_ref, t1_ref, c1b_ref, t2_ref, c2b_ref,
                w1_ref, b1_ref, w2_ref, b2_ref, w3_ref, b3_ref, o_ref):
    B = x_ref.shape[0]

    acc = None
    for kh in range(5):
        xs = x_ref[:, kh:kh + 28, :].reshape(B * 28, 96)
        d = jnp.dot(xs, t1_ref[kh], preferred_element_type=jnp.float32)
        acc = d if acc is None else acc + d
    y = jnp.maximum(acc + c1b_ref[...], 0.0).reshape(B, 14, 2, 256)
    y = jnp.maximum(y[:, :, 0, :], y[:, :, 1, :])
    x2 = jnp.maximum(y[:, :, :128], y[:, :, 128:])

    acc2 = None
    for kh in range(5):
        xs2 = x2[:, kh:kh + 10, :].reshape(B * 10, 128)
        d = jnp.dot(xs2, t2_ref[kh], preferred_element_type=jnp.float32)
        acc2 = d if acc2 is None else acc2 + d
    y2 = jnp.maximum(acc2 + c2b_ref[...], 0.0).reshape(B, 5, 2, 256)
    y2 = jnp.maximum(y2[:, :, 0, :], y2[:, :, 1, :])
    feat = jnp.maximum(y2[:, :, :128], y2[:, :, 128:])

    h = None
    for r in range(5):
        d = jnp.dot(feat[:, r, :], w1_ref[r],
                    preferred_element_type=jnp.float32)
        h = d if h is None else h + d
    h = jnp.maximum(h + b1_ref[...], 0.0)
    h = jnp.dot(h, w2_ref[...], preferred_element_type=jnp.float32)
    h = jnp.maximum(h + b2_ref[...], 0.0)
    h = jnp.dot(h, w3_ref[...], preferred_element_type=jnp.float32)
    o_ref[...] = h + b3_ref[...]


def _build_tables(conv1_w, conv1_b, conv2_w, conv2_b,
                  fc1_w, fc1_b, fc2_w, fc2_b, fc3_w, fc3_b):
    f32 = jnp.float32

    wt1 = conv1_w.astype(f32).transpose(1, 2, 3, 0)
    ow = jnp.arange(28)
    t1 = jnp.zeros((5, 32, 3, 2, 14, 6), f32)
    for kw in range(5):
        vals = jnp.broadcast_to(wt1[:, :, kw, :].transpose(1, 0, 2)[None],
                                (28, 5, 3, 6))
        t1 = t1.at[:, ow + kw, :, ow % 2, ow // 2, :].set(vals)
    t1 = jnp.pad(t1.reshape(5, 96, 2, 84), ((0, 0), (0, 0), (0, 0), (0, 44)))
    t1 = t1.reshape(5, 96, 256)

    wt2 = conv2_w.astype(f32).transpose(1, 2, 3, 0)
    ow2 = jnp.arange(10)
    t2 = jnp.zeros((5, 14, 6, 2, 5, 16), f32)
    for kw in range(5):
        vals = jnp.broadcast_to(wt2[:, :, kw, :].transpose(1, 0, 2)[None],
                                (10, 5, 6, 16))
        t2 = t2.at[:, ow2 + kw, :, ow2 % 2, ow2 // 2, :].set(vals)
    t2 = jnp.pad(t2.reshape(5, 84, 2, 80),
                 ((0, 0), (0, 44), (0, 0), (0, 48)))
    t2 = t2.reshape(5, 128, 256)

    half1 = jnp.pad(jnp.tile(conv1_b.astype(f32), 14), (0, 44))
    c1b = jnp.concatenate([half1, half1]).reshape(1, 256)
    half2 = jnp.pad(jnp.tile(conv2_b.astype(f32), 5), (0, 48))
    c2b = jnp.concatenate([half2, half2]).reshape(1, 256)

    w1 = (fc1_w.astype(f32).reshape(16, 5, 5, 120)
          .transpose(1, 2, 0, 3).reshape(5, 80, 120))
    w1 = jnp.pad(w1, ((0, 0), (0, 48), (0, 8)))
    b1 = jnp.pad(fc1_b.astype(f32), (0, 8)).reshape(1, 128)
    w2 = jnp.pad(fc2_w.astype(f32), ((0, 8), (0, 44)))
    b2 = jnp.pad(fc2_b.astype(f32), (0, 44)).reshape(1, 128)
    w3 = jnp.pad(fc3_w.astype(f32), ((0, 44), (0, 118)))
    b3 = jnp.pad(fc3_b.astype(f32), (0, 118)).reshape(1, 128)
    return t1, c1b, t2, c2b, w1, b1, w2, b2, w3, b3


def kernel(x, conv1_w, conv1_b, conv2_w, conv2_b,
           fc1_w, fc1_b, fc2_w, fc2_b, fc3_w, fc3_b):
    N = x.shape[0]
    B = 256
    while N % B:
        B //= 2
    tables = _build_tables(conv1_w, conv1_b, conv2_w, conv2_b,
                           fc1_w, fc1_b, fc2_w, fc2_b, fc3_w, fc3_b)
    xt = x.astype(jnp.float32).transpose(0, 2, 3, 1).reshape(N, 32, 96)
    out = pl.pallas_call(
        _lenet_body,
        out_shape=jax.ShapeDtypeStruct((N, 128), jnp.float32),
        grid=(N // B,),
        in_specs=[
            pl.BlockSpec((B, 32, 96), lambda i: (i, 0, 0)),
            pl.BlockSpec((5, 96, 256), lambda i: (0, 0, 0)),
            pl.BlockSpec((1, 256), lambda i: (0, 0)),
            pl.BlockSpec((5, 128, 256), lambda i: (0, 0, 0)),
            pl.BlockSpec((1, 256), lambda i: (0, 0)),
            pl.BlockSpec((5, 128, 128), lambda i: (0, 0, 0)),
            pl.BlockSpec((1, 128), lambda i: (0, 0)),
            pl.BlockSpec((128, 128), lambda i: (0, 0)),
            pl.BlockSpec((1, 128), lambda i: (0, 0)),
            pl.BlockSpec((128, 128), lambda i: (0, 0)),
            pl.BlockSpec((1, 128), lambda i: (0, 0)),
        ],
        out_specs=pl.BlockSpec((B, 128), lambda i: (i, 0)),
        compiler_params=pltpu.CompilerParams(
            dimension_semantics=("parallel",),
            vmem_limit_bytes=100 * 1024 * 1024,
        ),
        cost_estimate=pl.CostEstimate(
            flops=16_000_000_000,
            transcendentals=0,
            bytes_accessed=x.size * 4 + N * 128 * 4,
        ),
    )(xt, *tables)
    return out[:, :10]


# speedup vs baseline: 1.5805x; 1.3103x over previous
import jax
import jax.numpy as jnp
from jax.experimental import pallas as pl
from jax.experimental.pallas import tpu as pltpu


def _lenet_body(x_ref, t1_ref, c1b_ref, t2_ref, c2b_ref,
                w1_ref, b1_ref, w2_ref, b2_ref, w3_ref, b3_ref, o_ref):
    B = x_ref.shape[1]
    x = x_ref[...]

    xs = jnp.concatenate([x[kh:kh + 28].reshape(28 * B, 128)
                          for kh in range(5)], axis=1)
    acc = jnp.dot(xs, t1_ref[...], preferred_element_type=jnp.float32)
    y = jnp.maximum(acc + c1b_ref[...], 0.0).astype(jnp.bfloat16)
    y = y.reshape(14, 2, B, 256)
    y = jnp.maximum(y[:, 0], y[:, 1])
    x2 = jnp.maximum(y[:, :, :128], y[:, :, 128:])

    xs2 = jnp.concatenate([x2[kh:kh + 10].reshape(10 * B, 128)
                           for kh in range(5)], axis=1)
    acc2 = jnp.dot(xs2, t2_ref[...], preferred_element_type=jnp.float32)
    y2 = jnp.maximum(acc2 + c2b_ref[...], 0.0).astype(jnp.bfloat16)
    y2 = y2.reshape(5, 2, B, 256)
    y2 = jnp.maximum(y2[:, 0], y2[:, 1])
    feat = jnp.maximum(y2[:, :, :128], y2[:, :, 128:])

    h = None
    for r in range(5):
        d = jnp.dot(feat[r], w1_ref[r], preferred_element_type=jnp.float32)
        h = d if h is None else h + d
    h = jnp.maximum(h + b1_ref[...], 0.0).astype(jnp.bfloat16)
    h = jnp.dot(h, w2_ref[...], preferred_element_type=jnp.float32)
    h = jnp.maximum(h + b2_ref[...], 0.0).astype(jnp.bfloat16)
    h = jnp.dot(h, w3_ref[...], preferred_element_type=jnp.float32)
    o_ref[...] = h + b3_ref[...]


def _build_tables(conv1_w, conv1_b, conv2_w, conv2_b,
                  fc1_w, fc1_b, fc2_w, fc2_b, fc3_w, fc3_b):
    f32 = jnp.float32

    wt1 = conv1_w.astype(f32).transpose(1, 2, 3, 0)
    ow = jnp.arange(28)
    t1 = jnp.zeros((5, 32, 3, 2, 14, 6), f32)
    for kw in range(5):
        vals = jnp.broadcast_to(wt1[:, :, kw, :].transpose(1, 0, 2)[None],
                                (28, 5, 3, 6))
        t1 = t1.at[:, ow + kw, :, ow % 2, ow // 2, :].set(vals)
    t1 = jnp.pad(t1.reshape(5, 96, 2, 84), ((0, 0), (0, 32), (0, 0), (0, 44)))
    t1 = t1.reshape(640, 256)

    wt2 = conv2_w.astype(f32).transpose(1, 2, 3, 0)
    ow2 = jnp.arange(10)
    t2 = jnp.zeros((5, 14, 6, 2, 5, 16), f32)
    for kw in range(5):
        vals = jnp.broadcast_to(wt2[:, :, kw, :].transpose(1, 0, 2)[None],
                                (10, 5, 6, 16))
        t2 = t2.at[:, ow2 + kw, :, ow2 % 2, ow2 // 2, :].set(vals)
    t2 = jnp.pad(t2.reshape(5, 84, 2, 80),
                 ((0, 0), (0, 44), (0, 0), (0, 48)))
    t2 = t2.reshape(640, 256)

    half1 = jnp.pad(jnp.tile(conv1_b.astype(f32), 14), (0, 44))
    c1b = jnp.concatenate([half1, half1]).reshape(1, 256)
    half2 = jnp.pad(jnp.tile(conv2_b.astype(f32), 5), (0, 48))
    c2b = jnp.concatenate([half2, half2]).reshape(1, 256)

    w1 = (fc1_w.astype(f32).reshape(16, 5, 5, 120)
          .transpose(1, 2, 0, 3).reshape(5, 80, 120))
    w1 = jnp.pad(w1, ((0, 0), (0, 48), (0, 8)))
    b1 = jnp.pad(fc1_b.astype(f32), (0, 8)).reshape(1, 128)
    w2 = jnp.pad(fc2_w.astype(f32), ((0, 8), (0, 44)))
    b2 = jnp.pad(fc2_b.astype(f32), (0, 44)).reshape(1, 128)
    w3 = jnp.pad(fc3_w.astype(f32), ((0, 44), (0, 118)))
    b3 = jnp.pad(fc3_b.astype(f32), (0, 118)).reshape(1, 128)
    bf16 = jnp.bfloat16
    return (t1.astype(bf16), c1b, t2.astype(bf16), c2b,
            w1.astype(bf16), b1, w2.astype(bf16), b2, w3.astype(bf16), b3)


def kernel(x, conv1_w, conv1_b, conv2_w, conv2_b,
           fc1_w, fc1_b, fc2_w, fc2_b, fc3_w, fc3_b):
    N = x.shape[0]
    B = 256
    while N % B:
        B //= 2
    tables = _build_tables(conv1_w, conv1_b, conv2_w, conv2_b,
                           fc1_w, fc1_b, fc2_w, fc2_b, fc3_w, fc3_b)
    xt = jnp.pad(x.astype(jnp.bfloat16).transpose(2, 0, 3, 1).reshape(32, N, 96),
                 ((0, 0), (0, 0), (0, 32)))
    out = pl.pallas_call(
        _lenet_body,
        out_shape=jax.ShapeDtypeStruct((N, 128), jnp.float32),
        grid=(N // B,),
        in_specs=[
            pl.BlockSpec((32, B, 128), lambda i: (0, i, 0)),
            pl.BlockSpec((640, 256), lambda i: (0, 0)),
            pl.BlockSpec((1, 256), lambda i: (0, 0)),
            pl.BlockSpec((640, 256), lambda i: (0, 0)),
            pl.BlockSpec((1, 256), lambda i: (0, 0)),
            pl.BlockSpec((5, 128, 128), lambda i: (0, 0, 0)),
            pl.BlockSpec((1, 128), lambda i: (0, 0)),
            pl.BlockSpec((128, 128), lambda i: (0, 0)),
            pl.BlockSpec((1, 128), lambda i: (0, 0)),
            pl.BlockSpec((128, 128), lambda i: (0, 0)),
            pl.BlockSpec((1, 128), lambda i: (0, 0)),
        ],
        out_specs=pl.BlockSpec((B, 128), lambda i: (i, 0)),
        compiler_params=pltpu.CompilerParams(
            dimension_semantics=("parallel",),
            vmem_limit_bytes=100 * 1024 * 1024,
        ),
        cost_estimate=pl.CostEstimate(
            flops=16_000_000_000,
            transcendentals=0,
            bytes_accessed=x.size * 4 + N * 128 * 4,
        ),
    )(xt, *tables)
    return out[:, :10]


# speedup vs baseline: 1.5948x; 1.0091x over previous
import jax
import jax.numpy as jnp
from jax.experimental import pallas as pl
from jax.experimental.pallas import tpu as pltpu


def _lenet_body(x_ref, t1_ref, c1b_ref, t2_ref, c2b_ref,
                w1_ref, b1_ref, w2_ref, b2_ref, w3_ref, b3_ref, o_ref):
    B = x_ref.shape[1]
    x = x_ref[...]

    xs = jnp.concatenate([x[kh:kh + 28].reshape(28 * B, 128)
                          for kh in range(5)], axis=1)
    acc = jnp.dot(xs, t1_ref[...], preferred_element_type=jnp.float32)
    y = jnp.maximum(acc + c1b_ref[...], 0.0).astype(jnp.bfloat16)
    y = y.reshape(14, 2, B, 256)
    y = jnp.maximum(y[:, 0], y[:, 1])
    x2 = jnp.maximum(y[:, :, :128], y[:, :, 128:])

    xs2 = jnp.concatenate([x2[kh:kh + 10].reshape(10 * B, 128)
                           for kh in range(5)], axis=1)
    acc2 = jnp.dot(xs2, t2_ref[...], preferred_element_type=jnp.float32)
    y2 = jnp.maximum(acc2 + c2b_ref[...], 0.0).astype(jnp.bfloat16)
    y2 = y2.reshape(5, 2, B, 256)
    y2 = jnp.maximum(y2[:, 0], y2[:, 1])
    feat = jnp.maximum(y2[:, :, :128], y2[:, :, 128:])

    h = None
    for r in range(5):
        d = jnp.dot(feat[r], w1_ref[r], preferred_element_type=jnp.float32)
        h = d if h is None else h + d
    h = jnp.maximum(h + b1_ref[...], 0.0).astype(jnp.bfloat16)
    h = jnp.dot(h, w2_ref[...], preferred_element_type=jnp.float32)
    h = jnp.maximum(h + b2_ref[...], 0.0).astype(jnp.bfloat16)
    h = jnp.dot(h, w3_ref[...], preferred_element_type=jnp.float32)
    o_ref[...] = h + b3_ref[...]


def _build_tables(conv1_w, conv1_b, conv2_w, conv2_b,
                  fc1_w, fc1_b, fc2_w, fc2_b, fc3_w, fc3_b):
    f32 = jnp.float32

    wt1 = conv1_w.astype(f32).transpose(1, 2, 3, 0)
    ow = jnp.arange(28)
    t1 = jnp.zeros((5, 32, 3, 2, 14, 6), f32)
    for kw in range(5):
        vals = jnp.broadcast_to(wt1[:, :, kw, :].transpose(1, 0, 2)[None],
                                (28, 5, 3, 6))
        t1 = t1.at[:, ow + kw, :, ow % 2, ow // 2, :].set(vals)
    t1 = jnp.pad(t1.reshape(5, 96, 2, 84), ((0, 0), (0, 32), (0, 0), (0, 44)))
    t1 = t1.reshape(640, 256)

    wt2 = conv2_w.astype(f32).transpose(1, 2, 3, 0)
    ow2 = jnp.arange(10)
    t2 = jnp.zeros((5, 14, 6, 2, 5, 16), f32)
    for kw in range(5):
        vals = jnp.broadcast_to(wt2[:, :, kw, :].transpose(1, 0, 2)[None],
                                (10, 5, 6, 16))
        t2 = t2.at[:, ow2 + kw, :, ow2 % 2, ow2 // 2, :].set(vals)
    t2 = jnp.pad(t2.reshape(5, 84, 2, 80),
                 ((0, 0), (0, 44), (0, 0), (0, 48)))
    t2 = t2.reshape(640, 256)

    half1 = jnp.pad(jnp.tile(conv1_b.astype(f32), 14), (0, 44))
    c1b = jnp.concatenate([half1, half1]).reshape(1, 256)
    half2 = jnp.pad(jnp.tile(conv2_b.astype(f32), 5), (0, 48))
    c2b = jnp.concatenate([half2, half2]).reshape(1, 256)

    w1 = (fc1_w.astype(f32).reshape(16, 5, 5, 120)
          .transpose(1, 2, 0, 3).reshape(5, 80, 120))
    w1 = jnp.pad(w1, ((0, 0), (0, 48), (0, 8)))
    b1 = jnp.pad(fc1_b.astype(f32), (0, 8)).reshape(1, 128)
    w2 = jnp.pad(fc2_w.astype(f32), ((0, 8), (0, 44)))
    b2 = jnp.pad(fc2_b.astype(f32), (0, 44)).reshape(1, 128)
    w3 = jnp.pad(fc3_w.astype(f32), ((0, 44), (0, 118)))
    b3 = jnp.pad(fc3_b.astype(f32), (0, 118)).reshape(1, 128)
    bf16 = jnp.bfloat16
    return (t1.astype(bf16), c1b, t2.astype(bf16), c2b,
            w1.astype(bf16), b1, w2.astype(bf16), b2, w3.astype(bf16), b3)


def kernel(x, conv1_w, conv1_b, conv2_w, conv2_b,
           fc1_w, fc1_b, fc2_w, fc2_b, fc3_w, fc3_b):
    N = x.shape[0]
    B = 512
    while N % B:
        B //= 2
    tables = _build_tables(conv1_w, conv1_b, conv2_w, conv2_b,
                           fc1_w, fc1_b, fc2_w, fc2_b, fc3_w, fc3_b)
    xt = jnp.pad(x.astype(jnp.bfloat16).transpose(2, 0, 3, 1).reshape(32, N, 96),
                 ((0, 0), (0, 0), (0, 32)))
    out = pl.pallas_call(
        _lenet_body,
        out_shape=jax.ShapeDtypeStruct((N, 128), jnp.float32),
        grid=(N // B,),
        in_specs=[
            pl.BlockSpec((32, B, 128), lambda i: (0, i, 0)),
            pl.BlockSpec((640, 256), lambda i: (0, 0)),
            pl.BlockSpec((1, 256), lambda i: (0, 0)),
            pl.BlockSpec((640, 256), lambda i: (0, 0)),
            pl.BlockSpec((1, 256), lambda i: (0, 0)),
            pl.BlockSpec((5, 128, 128), lambda i: (0, 0, 0)),
            pl.BlockSpec((1, 128), lambda i: (0, 0)),
            pl.BlockSpec((128, 128), lambda i: (0, 0)),
            pl.BlockSpec((1, 128), lambda i: (0, 0)),
            pl.BlockSpec((128, 128), lambda i: (0, 0)),
            pl.BlockSpec((1, 128), lambda i: (0, 0)),
        ],
        out_specs=pl.BlockSpec((B, 128), lambda i: (i, 0)),
        compiler_params=pltpu.CompilerParams(
            dimension_semantics=("parallel",),
            vmem_limit_bytes=100 * 1024 * 1024,
        ),
        cost_estimate=pl.CostEstimate(
            flops=16_000_000_000,
            transcendentals=0,
            bytes_accessed=x.size * 4 + N * 128 * 4,
        ),
    )(xt, *tables)
    return out[:, :10]


# speedup vs baseline: 1.7231x; 1.0804x over previous
import jax
import jax.numpy as jnp
from jax.experimental import pallas as pl
from jax.experimental.pallas import tpu as pltpu


def _lenet_body(x_ref, t1_ref, c1b_ref, t2_ref, c2b_ref,
                w1_ref, b1_ref, w2_ref, b2_ref, w3_ref, b3_ref, o_ref):
    B = x_ref.shape[1]
    x = x_ref[...]
    x = jnp.pad(x, ((0, 0), (0, 0), (0, 32)))

    xs = jnp.concatenate([x[kh:kh + 28].reshape(28 * B, 128)
                          for kh in range(5)], axis=1)
    acc = jnp.dot(xs, t1_ref[...], preferred_element_type=jnp.float32)
    y = jnp.maximum(acc + c1b_ref[...], 0.0).astype(jnp.bfloat16)
    y = y.reshape(14, 2, B, 256)
    y = jnp.maximum(y[:, 0], y[:, 1])
    x2 = jnp.maximum(y[:, :, :128], y[:, :, 128:])

    xs2 = jnp.concatenate([x2[kh:kh + 10].reshape(10 * B, 128)
                           for kh in range(5)], axis=1)
    acc2 = jnp.dot(xs2, t2_ref[...], preferred_element_type=jnp.float32)
    y2 = jnp.maximum(acc2 + c2b_ref[...], 0.0).astype(jnp.bfloat16)
    y2 = y2.reshape(5, 2, B, 256)
    y2 = jnp.maximum(y2[:, 0], y2[:, 1])
    feat = jnp.maximum(y2[:, :, :128], y2[:, :, 128:])

    h = None
    for r in range(5):
        d = jnp.dot(feat[r], w1_ref[r], preferred_element_type=jnp.float32)
        h = d if h is None else h + d
    h = jnp.maximum(h + b1_ref[...], 0.0).astype(jnp.bfloat16)
    h = jnp.dot(h, w2_ref[...], preferred_element_type=jnp.float32)
    h = jnp.maximum(h + b2_ref[...], 0.0).astype(jnp.bfloat16)
    h = jnp.dot(h, w3_ref[...], preferred_element_type=jnp.float32)
    o_ref[...] = h + b3_ref[...]


def _build_tables(conv1_w, conv1_b, conv2_w, conv2_b,
                  fc1_w, fc1_b, fc2_w, fc2_b, fc3_w, fc3_b):
    f32 = jnp.float32

    wt1 = conv1_w.astype(f32).transpose(1, 2, 3, 0)
    ow = jnp.arange(28)
    t1 = jnp.zeros((5, 32, 3, 2, 14, 6), f32)
    for kw in range(5):
        vals = jnp.broadcast_to(wt1[:, :, kw, :].transpose(1, 0, 2)[None],
                                (28, 5, 3, 6))
        t1 = t1.at[:, ow + kw, :, ow % 2, ow // 2, :].set(vals)
    t1 = jnp.pad(t1.reshape(5, 96, 2, 84), ((0, 0), (0, 32), (0, 0), (0, 44)))
    t1 = t1.reshape(640, 256)

    wt2 = conv2_w.astype(f32).transpose(1, 2, 3, 0)
    ow2 = jnp.arange(10)
    t2 = jnp.zeros((5, 14, 6, 2, 5, 16), f32)
    for kw in range(5):
        vals = jnp.broadcast_to(wt2[:, :, kw, :].transpose(1, 0, 2)[None],
                                (10, 5, 6, 16))
        t2 = t2.at[:, ow2 + kw, :, ow2 % 2, ow2 // 2, :].set(vals)
    t2 = jnp.pad(t2.reshape(5, 84, 2, 80),
                 ((0, 0), (0, 44), (0, 0), (0, 48)))
    t2 = t2.reshape(640, 256)

    half1 = jnp.pad(jnp.tile(conv1_b.astype(f32), 14), (0, 44))
    c1b = jnp.concatenate([half1, half1]).reshape(1, 256)
    half2 = jnp.pad(jnp.tile(conv2_b.astype(f32), 5), (0, 48))
    c2b = jnp.concatenate([half2, half2]).reshape(1, 256)

    w1 = (fc1_w.astype(f32).reshape(16, 5, 5, 120)
          .transpose(1, 2, 0, 3).reshape(5, 80, 120))
    w1 = jnp.pad(w1, ((0, 0), (0, 48), (0, 8)))
    b1 = jnp.pad(fc1_b.astype(f32), (0, 8)).reshape(1, 128)
    w2 = jnp.pad(fc2_w.astype(f32), ((0, 8), (0, 44)))
    b2 = jnp.pad(fc2_b.astype(f32), (0, 44)).reshape(1, 128)
    w3 = jnp.pad(fc3_w.astype(f32), ((0, 44), (0, 118)))
    b3 = jnp.pad(fc3_b.astype(f32), (0, 118)).reshape(1, 128)
    bf16 = jnp.bfloat16
    return (t1.astype(bf16), c1b, t2.astype(bf16), c2b,
            w1.astype(bf16), b1, w2.astype(bf16), b2, w3.astype(bf16), b3)


def kernel(x, conv1_w, conv1_b, conv2_w, conv2_b,
           fc1_w, fc1_b, fc2_w, fc2_b, fc3_w, fc3_b):
    N = x.shape[0]
    B = 512
    while N % B:
        B //= 2
    tables = _build_tables(conv1_w, conv1_b, conv2_w, conv2_b,
                           fc1_w, fc1_b, fc2_w, fc2_b, fc3_w, fc3_b)
    xt = x.astype(jnp.bfloat16).transpose(2, 0, 3, 1).reshape(32, N, 96)
    out = pl.pallas_call(
        _lenet_body,
        out_shape=jax.ShapeDtypeStruct((N, 128), jnp.float32),
        grid=(N // B,),
        in_specs=[
            pl.BlockSpec((32, B, 96), lambda i: (0, i, 0)),
            pl.BlockSpec((640, 256), lambda i: (0, 0)),
            pl.BlockSpec((1, 256), lambda i: (0, 0)),
            pl.BlockSpec((640, 256), lambda i: (0, 0)),
            pl.BlockSpec((1, 256), lambda i: (0, 0)),
            pl.BlockSpec((5, 128, 128), lambda i: (0, 0, 0)),
            pl.BlockSpec((1, 128), lambda i: (0, 0)),
            pl.BlockSpec((128, 128), lambda i: (0, 0)),
            pl.BlockSpec((1, 128), lambda i: (0, 0)),
            pl.BlockSpec((128, 128), lambda i: (0, 0)),
            pl.BlockSpec((1, 128), lambda i: (0, 0)),
        ],
        out_specs=pl.BlockSpec((B, 128), lambda i: (i, 0)),
        compiler_params=pltpu.CompilerParams(
            dimension_semantics=("parallel",),
            vmem_limit_bytes=100 * 1024 * 1024,
        ),
        cost_estimate=pl.CostEstimate(
            flops=16_000_000_000,
            transcendentals=0,
            bytes_accessed=x.size * 4 + N * 128 * 4,
        ),
    )(xt, *tables)
    return out[:, :10]
